# baseline (speedup 1.0000x reference)
import functools

import jax
import jax.numpy as jnp
from jax import lax
from jax.experimental import pallas as pl
from jax.experimental.pallas import tpu as pltpu
from jax.experimental.pallas import tpu_sc as plsc


def _sc_gather_t(indices, table_t):
    D, V = table_t.shape
    B, = indices.shape
    info = plsc.get_sparse_core_info()
    NC, NS = info.num_cores, info.num_subcores
    NW = NC * NS
    b_per_w = B // NW
    CHUNK = 4
    n_chunks = b_per_w // CHUNK

    mesh = plsc.VectorSubcoreMesh(core_axis_name="c", subcore_axis_name="s")

    @functools.partial(
        pl.kernel,
        mesh=mesh,
        compiler_params=pltpu.CompilerParams(needs_layout_passes=False),
        out_type=jax.ShapeDtypeStruct((B, D), jnp.float32),
        scratch_types=[
            pltpu.VMEM((b_per_w + 16,), jnp.int32),
            pltpu.VMEM((D, 2 * 128 * CHUNK), jnp.float32),
            pltpu.VMEM((b_per_w, D), jnp.float32),
            pltpu.SemaphoreType.DMA,
            pltpu.SemaphoreType.DMA,
        ],
    )
    def gather_k(tab_hbm, idx_hbm, out_hbm, idx_s, blk_v, stage_v, sem0, sem1):
        wid = lax.axis_index("s") * NC + lax.axis_index("c")
        base = wid * b_per_w
        pltpu.sync_copy(idx_hbm.at[pl.ds(base, b_per_w)],
                        idx_s.at[pl.ds(0, b_per_w)])

        rows0 = lax.iota(jnp.int32, 16)
        rows1 = rows0 + 16

        def fire(c, sem):
            half = lax.rem(c, 2)
            vec = idx_s[pl.ds(c * CHUNK, 16)]
            off = (vec >> 7) << 7
            for j in range(CHUNK):
                pltpu.async_copy(
                    tab_hbm.at[:, pl.ds(pl.multiple_of(off[j], 128), 128)],
                    blk_v.at[:, pl.ds(half * (128 * CHUNK) + 128 * j, 128)],
                    sem,
                )

        def drain(sem):
            for _ in range(CHUNK):
                pltpu.make_async_copy(
                    tab_hbm.at[:, pl.ds(0, 128)],
                    blk_v.at[:, pl.ds(0, 128)],
                    sem,
                ).wait()

        def extract(c):
            half = lax.rem(c, 2)
            vec = idx_s[pl.ds(c * CHUNK, 16)]
            rem = vec - ((vec >> 7) << 7)
            for j in range(CHUNK):
                col = half * (128 * CHUNK) + 128 * j + rem[j]
                cols = jnp.full((16,), col, jnp.int32)
                v0 = plsc.load_gather(blk_v, [rows0, cols])
                v1 = plsc.load_gather(blk_v, [rows1, cols])
                srow = jnp.full((16,), c * CHUNK + j, jnp.int32)
                plsc.store_scatter(stage_v, [srow, rows0], v0)
                plsc.store_scatter(stage_v, [srow, rows1], v1)

        fire(0, sem0)

        def body(c, _):
            sa = lax.rem(c, 2)

            @pl.when(sa == 0)
            def _():
                fire(c + 1, sem1)
                drain(sem0)

            @pl.when(sa == 1)
            def _():
                fire(c + 1, sem0)
                drain(sem1)

            extract(c)
            return ()

        lax.fori_loop(0, n_chunks - 1, body, (), unroll=False)
        last = n_chunks - 1

        @pl.when(lax.rem(last, 2) == 0)
        def _():
            drain(sem0)

        @pl.when(lax.rem(last, 2) == 1)
        def _():
            drain(sem1)

        extract(last)
        pltpu.sync_copy(stage_v, out_hbm.at[pl.ds(base, b_per_w)])

    return gather_k(table_t, indices)


def _tc_mlp_t(x, W1, b1, W2, b2):
    B, D = x.shape
    H = W1.shape[1]
    O = W2.shape[1]
    BLK = 2048
    grid = (B // BLK,)

    def body(x_ref, w1_ref, b1_ref, w2_ref, b2_ref, o_ref):
        h = jnp.dot(x_ref[...], w1_ref[...],
                    preferred_element_type=jnp.float32) + b1_ref[...]
        o_t = lax.dot_general(w2_ref[...], h,
                              (((0,), (1,)), ((), ())),
                              preferred_element_type=jnp.float32)
        o_ref[...] = o_t + b2_ref[...]

    return pl.pallas_call(
        body,
        grid=grid,
        in_specs=[
            pl.BlockSpec((BLK, D), lambda i: (i, 0)),
            pl.BlockSpec((D, H), lambda i: (0, 0)),
            pl.BlockSpec((1, H), lambda i: (0, 0)),
            pl.BlockSpec((H, O), lambda i: (0, 0)),
            pl.BlockSpec((O, 1), lambda i: (0, 0)),
        ],
        out_specs=pl.BlockSpec((O, BLK), lambda i: (0, i)),
        out_shape=jax.ShapeDtypeStruct((O, B), jnp.float32),
    )(x, W1, b1.reshape(1, H), W2, b2.reshape(O, 1))


def kernel(indices, table, W1, b1, W2, b2):
    idx = indices.astype(jnp.int32)
    gathered = _sc_gather_t(idx, table.T)
    return _tc_mlp_t(gathered, W1, b1, W2, b2).T


# speedup vs baseline: 1.0575x; 1.0575x over previous
import functools

import jax
import jax.numpy as jnp
from jax import lax
from jax.experimental import pallas as pl
from jax.experimental.pallas import tpu as pltpu
from jax.experimental.pallas import tpu_sc as plsc


def _sc_gather_t(indices, table_t):
    D, V = table_t.shape
    B, = indices.shape
    info = plsc.get_sparse_core_info()
    NC, NS = info.num_cores, info.num_subcores
    NW = NC * NS
    b_per_w = B // NW
    CHUNK = 8
    n_chunks = b_per_w // CHUNK

    mesh = plsc.VectorSubcoreMesh(core_axis_name="c", subcore_axis_name="s")

    @functools.partial(
        pl.kernel,
        mesh=mesh,
        compiler_params=pltpu.CompilerParams(needs_layout_passes=False),
        out_type=jax.ShapeDtypeStruct((B * D,), jnp.float32),
        scratch_types=[
            pltpu.VMEM((b_per_w + 16,), jnp.int32),
            pltpu.VMEM((D, 2 * 128 * CHUNK), jnp.float32),
            pltpu.VMEM((b_per_w * D,), jnp.float32),
            pltpu.SemaphoreType.DMA,
            pltpu.SemaphoreType.DMA,
        ],
    )
    def gather_k(tab_hbm, idx_hbm, out_hbm, idx_s, blk_v, stage_v, sem0, sem1):
        wid = lax.axis_index("s") * NC + lax.axis_index("c")
        base = wid * b_per_w
        pltpu.sync_copy(idx_hbm.at[pl.ds(base, b_per_w)],
                        idx_s.at[pl.ds(0, b_per_w)])

        rows0 = lax.iota(jnp.int32, 16)
        rows1 = rows0 + 16

        def fire(c, sem):
            half = lax.rem(c, 2)
            vec = idx_s[pl.ds(c * CHUNK, 16)]
            off = (vec >> 7) << 7
            for j in range(CHUNK):
                pltpu.async_copy(
                    tab_hbm.at[:, pl.ds(pl.multiple_of(off[j], 128), 128)],
                    blk_v.at[:, pl.ds(half * (128 * CHUNK) + 128 * j, 128)],
                    sem,
                )

        def drain(sem):
            for _ in range(CHUNK):
                pltpu.make_async_copy(
                    tab_hbm.at[:, pl.ds(0, 128)],
                    blk_v.at[:, pl.ds(0, 128)],
                    sem,
                ).wait()

        def extract(c):
            half = lax.rem(c, 2)
            vec = idx_s[pl.ds(c * CHUNK, 16)]
            rem = vec - ((vec >> 7) << 7)
            for j in range(CHUNK):
                col = half * (128 * CHUNK) + 128 * j + rem[j]
                cols = jnp.full((16,), col, jnp.int32)
                v0 = plsc.load_gather(blk_v, [rows0, cols])
                v1 = plsc.load_gather(blk_v, [rows1, cols])
                s = (c * CHUNK + j) * D
                plsc.store_scatter(stage_v, [s + rows0], v0)
                plsc.store_scatter(stage_v, [s + 16 + rows0], v1)

        fire(0, sem0)

        def body(c, _):
            sa = lax.rem(c, 2)

            @pl.when(sa == 0)
            def _():
                fire(c + 1, sem1)
                drain(sem0)

            @pl.when(sa == 1)
            def _():
                fire(c + 1, sem0)
                drain(sem1)

            extract(c)
            return ()

        lax.fori_loop(0, n_chunks - 1, body, (), unroll=False)
        last = n_chunks - 1

        @pl.when(lax.rem(last, 2) == 0)
        def _():
            drain(sem0)

        @pl.when(lax.rem(last, 2) == 1)
        def _():
            drain(sem1)

        extract(last)
        pltpu.sync_copy(stage_v, out_hbm.at[pl.ds(base * D, b_per_w * D)])

    return gather_k(table_t, indices)


def _tc_mlp_t(x, W1, b1, W2, b2):
    B, D = x.shape
    H = W1.shape[1]
    O = W2.shape[1]
    BLK = 2048
    grid = (B // BLK,)

    def body(x_ref, w1_ref, b1_ref, w2_ref, b2_ref, o_ref):
        h = jnp.dot(x_ref[...], w1_ref[...],
                    preferred_element_type=jnp.float32) + b1_ref[...]
        o_t = lax.dot_general(w2_ref[...], h,
                              (((0,), (1,)), ((), ())),
                              preferred_element_type=jnp.float32)
        o_ref[...] = o_t + b2_ref[...]

    return pl.pallas_call(
        body,
        grid=grid,
        in_specs=[
            pl.BlockSpec((BLK, D), lambda i: (i, 0)),
            pl.BlockSpec((D, H), lambda i: (0, 0)),
            pl.BlockSpec((1, H), lambda i: (0, 0)),
            pl.BlockSpec((H, O), lambda i: (0, 0)),
            pl.BlockSpec((O, 1), lambda i: (0, 0)),
        ],
        out_specs=pl.BlockSpec((O, BLK), lambda i: (0, i)),
        out_shape=jax.ShapeDtypeStruct((O, B), jnp.float32),
    )(x, W1, b1.reshape(1, H), W2, b2.reshape(O, 1))


def kernel(indices, table, W1, b1, W2, b2):
    idx = indices.astype(jnp.int32)
    B = idx.shape[0]
    D = table.shape[1]
    gathered = _sc_gather_t(idx, table.T).reshape(B, D)
    return _tc_mlp_t(gathered, W1, b1, W2, b2).T
